# baseline (speedup 1.0000x reference)
import functools

import jax
import jax.numpy as jnp
from jax import lax
from jax.experimental import pallas as pl
from jax.experimental.pallas import tpu as pltpu
from jax.experimental.pallas import tpu_sc as plsc

BATCH = 4096
HIST = 50
D = 64
NC = 2
NS = 16
NW = NC * NS
BPW = BATCH // NW
LANES = 16
PAIRS = 500000

_sc_mesh = plsc.VectorSubcoreMesh(core_axis_name="c", subcore_axis_name="s")


@functools.partial(
    pl.kernel,
    out_type=(
        jax.ShapeDtypeStruct((BATCH, 2 * D), jnp.float32),
        jax.ShapeDtypeStruct((BATCH, 2 * D), jnp.float32),
    ),
    mesh=_sc_mesh,
    scratch_types=[
        pltpu.VMEM((HIST, BPW), jnp.int32),
        pltpu.VMEM((HIST, BPW), jnp.int32),
        pltpu.VMEM((HIST, BPW), jnp.int32),
        pltpu.VMEM((BPW, 2 * D), jnp.float32),
        pltpu.VMEM((BPW, 2 * D), jnp.float32),
        pltpu.SemaphoreType.DMA,
        pltpu.SemaphoreType.DMA,
    ],
)
def _pool(xt_hbm, tbl2_hbm, outa_hbm, outb_hbm,
          idx_v, ga_v, gb_v, acca_v, accb_v, sem_i, sem_g):
    wid = lax.axis_index("s") * NC + lax.axis_index("c")
    base = wid * BPW

    idx_cp = pltpu.async_copy(xt_hbm.at[:, pl.ds(base, BPW)], idx_v, sem_i)

    zero = jnp.zeros((LANES,), jnp.float32)

    def zbody(i, carry):
        for k in range(2 * D // LANES):
            sl = pl.ds(k * LANES, LANES)
            acca_v[i, sl] = zero
            accb_v[i, sl] = zero
        return carry

    lax.fori_loop(0, BPW, zbody, 0)
    idx_cp.wait()

    def pbody(l, carry):
        for k in range(BPW // LANES):
            sl = pl.ds(k * LANES, LANES)
            v = idx_v[l, sl]
            p = v & 1
            pid = lax.shift_right_logical(v, 1)
            ga_v[l, sl] = pid * (1 - p)
            gb_v[l, sl] = pid * p
        return carry

    lax.fori_loop(0, HIST, pbody, 0)

    def fire(l, carry):
        pltpu.async_copy(tbl2_hbm.at[ga_v.at[l]], acca_v, sem_g, add=True)
        pltpu.async_copy(tbl2_hbm.at[gb_v.at[l]], accb_v, sem_g, add=True)
        return carry

    lax.fori_loop(0, HIST, fire, 0)

    def drain(l, carry):
        pltpu.make_async_copy(tbl2_hbm.at[ga_v.at[0]], acca_v, sem_g).wait()
        pltpu.make_async_copy(tbl2_hbm.at[gb_v.at[0]], accb_v, sem_g).wait()
        return carry

    lax.fori_loop(0, HIST, drain, 0)
    pltpu.sync_copy(acca_v, outa_hbm.at[pl.ds(base, BPW)])
    pltpu.sync_copy(accb_v, outb_hbm.at[pl.ds(base, BPW)])


def _tr_body(x_ref, o_ref):
    o_ref[...] = x_ref[...].T


_transpose = pl.pallas_call(
    _tr_body,
    out_shape=jax.ShapeDtypeStruct((HIST, BATCH), jnp.int32),
)


def _linear_body(a_ref, b2_ref, x_ref, t01_ref, w_ref, b_ref, o_ref):
    cnt_odd = jnp.sum((x_ref[...] & 1).astype(jnp.float32), axis=1,
                      keepdims=True)
    cnt_even = jnp.float32(HIST) - cnt_odd
    sums = (a_ref[:, :D] + b2_ref[:, D:]
            - cnt_odd * t01_ref[0:1, :] - cnt_even * t01_ref[1:2, :])
    pooled = sums * (1.0 / HIST)
    o_ref[...] = lax.dot_general(
        pooled, w_ref[...],
        dimension_numbers=(((1,), (1,)), ((), ())),
        preferred_element_type=jnp.float32,
    ) + b_ref[...]


_linear = pl.pallas_call(
    _linear_body,
    out_shape=jax.ShapeDtypeStruct((BATCH, D), jnp.float32),
)


def kernel(x, table, W, b):
    xt = _transpose(x)
    tbl2 = table.reshape(PAIRS, 2 * D)
    acc_a, acc_b = _pool(xt, tbl2)
    return _linear(acc_a, acc_b, x, table[0:2], W, b.reshape(1, D))
